# baseline (speedup 1.0000x reference)
import functools

import jax
import jax.numpy as jnp
from jax import lax
from jax.experimental import pallas as pl
from jax.experimental.pallas import tpu as pltpu
from jax.experimental.pallas import tpu_sc as plsc

B, N, C = 8, 4096, 256
NPOINT, NSAMPLE = 1024, 32

NC, NS, L = 2, 16, 16
W = 2 * L
NW = NC * NS
PPW = (B * NPOINT) // NW
CHUNK = 4
ROWS = CHUNK * NSAMPLE
NCHUNK = PPW // CHUNK
OUTCHUNK = 8
NITER = PPW // (2 * OUTCHUNK)
CGB = C // W

_mesh = plsc.VectorSubcoreMesh(core_axis_name="c", subcore_axis_name="s")


@functools.partial(
    pl.kernel,
    out_type=jax.ShapeDtypeStruct((B * NPOINT, C // 2), jnp.int32),
    mesh=_mesh,
    scratch_types=[
        pltpu.VMEM((NCHUNK, ROWS), jnp.int32),
        pltpu.VMEM((ROWS, C // 2), jnp.int32),
        pltpu.VMEM((ROWS, C // 2), jnp.int32),
        pltpu.VMEM((OUTCHUNK, C // 2), jnp.int32),
        pltpu.VMEM((OUTCHUNK, C // 2), jnp.int32),
        pltpu.SemaphoreType.DMA,
        pltpu.SemaphoreType.DMA,
        pltpu.SemaphoreType.DMA,
        pltpu.SemaphoreType.DMA,
    ],
)
def _pool(feat_hbm, idx_hbm, out_hbm, idx_v, rows_a, rows_b, out_a, out_b,
          sem_a, sem_b, sem_oa, sem_ob):
    wid = lax.axis_index("s") * NC + lax.axis_index("c")
    base = wid * PPW
    boff = (base // NPOINT) * N

    pltpu.sync_copy(idx_hbm.at[wid], idx_v)

    def _rebase(r, carry):
        for t in range(ROWS // L):
            idx_v[r, pl.ds(t * L, L)] = idx_v[r, pl.ds(t * L, L)] + boff
        return carry
    lax.fori_loop(0, NCHUNK, _rebase, None)

    def _gather(c, rows_v, sem):
        return pltpu.make_async_copy(feat_hbm.at[idx_v.at[c]], rows_v, sem)

    def _owrite(k8, out_v, sem):
        return pltpu.make_async_copy(
            out_v, out_hbm.at[pl.ds(base + k8 * OUTCHUNK, OUTCHUNK)], sem)

    def _compute(rows_v, out_v, h):
        def _ld(r, g):
            return rows_v[r, pl.ds(g * L, L)]

        def _colgroup(g, carry):
            for p in range(CHUNK):
                r0 = p * NSAMPLE
                v0 = [_ld(r0 + t, g) for t in range(4)]
                hi = [v for v in v0]
                lo = [v << 16 for v in v0]
                for s in range(4, NSAMPLE, 4):
                    for t in range(4):
                        v = _ld(r0 + s + t, g)
                        hi[t] = jnp.maximum(hi[t], v)
                        lo[t] = jnp.maximum(lo[t], v << 16)
                a_hi = jnp.maximum(jnp.maximum(hi[0], hi[1]),
                                   jnp.maximum(hi[2], hi[3]))
                a_lo = jnp.maximum(jnp.maximum(lo[0], lo[1]),
                                   jnp.maximum(lo[2], lo[3]))
                word = (a_hi & jnp.int32(-65536)) | ((a_lo >> 16) & 0xFFFF)
                out_v[h * CHUNK + p, pl.ds(g * L, L)] = word
            return carry
        lax.fori_loop(0, CGB, _colgroup, None)

    _gather(0, rows_a, sem_a).start()
    _gather(1, rows_b, sem_b).start()

    def _step(k, carry):
        c0 = k * 4
        k8 = k * 2
        _gather(c0, rows_a, sem_a).wait()

        @pl.when(k > 0)
        def _():
            _owrite(k8 - 2, out_a, sem_oa).wait()
        _compute(rows_a, out_a, 0)
        _gather(c0 + 2, rows_a, sem_a).start()

        _gather(c0 + 1, rows_b, sem_b).wait()
        _compute(rows_b, out_a, 1)
        _gather(c0 + 3, rows_b, sem_b).start()
        _owrite(k8, out_a, sem_oa).start()

        _gather(c0 + 2, rows_a, sem_a).wait()

        @pl.when(k > 0)
        def _():
            _owrite(k8 - 1, out_b, sem_ob).wait()
        _compute(rows_a, out_b, 0)

        @pl.when(k < NITER - 1)
        def _():
            _gather(c0 + 4, rows_a, sem_a).start()

        _gather(c0 + 3, rows_b, sem_b).wait()
        _compute(rows_b, out_b, 1)

        @pl.when(k < NITER - 1)
        def _():
            _gather(c0 + 5, rows_b, sem_b).start()
        _owrite(k8 + 1, out_b, sem_ob).start()
        return carry

    lax.fori_loop(0, NITER, _step, None)
    _owrite(2 * NITER - 2, out_a, sem_oa).wait()
    _owrite(2 * NITER - 1, out_b, sem_ob).wait()


def _encode(x_bf16_bits):
    return jnp.where(x_bf16_bits >= 0, x_bf16_bits,
                     x_bf16_bits ^ jnp.int16(0x7FFF))


def kernel(features, coarse_map):
    feat_bf = features.astype(jnp.bfloat16).reshape(B * N, C)
    keys = _encode(lax.bitcast_convert_type(feat_bf, jnp.int16))
    feat_i32 = lax.bitcast_convert_type(keys.reshape(B * N, C // 2, 2),
                                        jnp.int32)
    idx_flat = coarse_map.reshape(NW, NCHUNK, ROWS)
    out = _pool(feat_i32, idx_flat)
    out_keys = lax.bitcast_convert_type(out, jnp.int16).reshape(B * NPOINT, C)
    out_bf = lax.bitcast_convert_type(_encode(out_keys), jnp.bfloat16)
    return out_bf.astype(jnp.float32).reshape(B, NPOINT, C)


# speedup vs baseline: 3.3012x; 3.3012x over previous
import functools

import jax
import jax.numpy as jnp
from jax import lax
from jax.experimental import pallas as pl
from jax.experimental.pallas import tpu as pltpu
from jax.experimental.pallas import tpu_sc as plsc

B, N, C = 8, 4096, 256
NPOINT, NSAMPLE = 1024, 32

NC, NS, L = 2, 16, 16
NW = NC * NS
PPW = (B * NPOINT) // NW
CHUNK = 2
ROWS = CHUNK * NSAMPLE
NCHUNK = PPW // CHUNK
NBUF = 4
OUTCHUNK = 8
CPG = OUTCHUNK // CHUNK
NITER = PPW // (2 * OUTCHUNK)
CG = C // L

_mesh = plsc.VectorSubcoreMesh(core_axis_name="c", subcore_axis_name="s")


@functools.partial(
    pl.kernel,
    out_type=jax.ShapeDtypeStruct((B * NPOINT, C), jnp.float32),
    mesh=_mesh,
    scratch_types=[
        pltpu.VMEM((NCHUNK, ROWS), jnp.int32),
        pltpu.VMEM((NBUF, ROWS, C), jnp.float32),
        pltpu.VMEM((OUTCHUNK, C), jnp.float32),
        pltpu.VMEM((OUTCHUNK, C), jnp.float32),
        pltpu.SemaphoreType.DMA,
        pltpu.SemaphoreType.DMA,
        pltpu.SemaphoreType.DMA,
        pltpu.SemaphoreType.DMA,
        pltpu.SemaphoreType.DMA,
        pltpu.SemaphoreType.DMA,
    ],
)
def _pool(feat_hbm, idx_hbm, out_hbm, idx_v, rows_v, out_a, out_b,
          sem_g0, sem_g1, sem_g2, sem_g3, sem_oa, sem_ob):
    wid = lax.axis_index("s") * NC + lax.axis_index("c")
    base = wid * PPW
    boff = (base // NPOINT) * N
    gsems = (sem_g0, sem_g1, sem_g2, sem_g3)

    pltpu.sync_copy(idx_hbm.at[wid], idx_v)

    def _rebase(r, carry):
        for t in range(ROWS // L):
            idx_v[r, pl.ds(t * L, L)] = idx_v[r, pl.ds(t * L, L)] + boff
        return carry
    lax.fori_loop(0, NCHUNK, _rebase, None)

    def _gather(c, slot):
        return pltpu.make_async_copy(
            feat_hbm.at[idx_v.at[c]], rows_v.at[slot], gsems[slot])

    def _owrite(og, out_v, sem):
        return pltpu.make_async_copy(
            out_v, out_hbm.at[pl.ds(base + og * OUTCHUNK, OUTCHUNK)], sem)

    def _compute(slot, out_v, orow):
        def _colgroup(g, carry):
            for p in range(CHUNK):
                r0 = p * NSAMPLE
                accs = [rows_v[slot, r0 + t, pl.ds(g * L, L)]
                        for t in range(4)]
                for s in range(4, NSAMPLE, 4):
                    for t in range(4):
                        accs[t] = jnp.maximum(
                            accs[t], rows_v[slot, r0 + s + t, pl.ds(g * L, L)])
                acc = jnp.maximum(jnp.maximum(accs[0], accs[1]),
                                  jnp.maximum(accs[2], accs[3]))
                out_v[orow + p, pl.ds(g * L, L)] = acc
            return carry
        lax.fori_loop(0, CG, _colgroup, None)

    for slot in range(NBUF):
        _gather(slot, slot).start()

    def _step(k, carry):
        c0 = k * 2 * CPG
        for half, (out_v, sem_o) in enumerate(
                ((out_a, sem_oa), (out_b, sem_ob))):
            og = k * 2 + half
            for j in range(CPG):
                off = half * CPG + j
                c = c0 + off
                slot = off % NBUF
                _gather(c, slot).wait()
                if j == 0:
                    @pl.when(og >= 2)
                    def _():
                        _owrite(og - 2, out_v, sem_o).wait()
                _compute(slot, out_v, j * CHUNK)
                if off < 2 * CPG - NBUF:
                    _gather(c + NBUF, slot).start()
                else:
                    @pl.when(k < NITER - 1)
                    def _():
                        _gather(c + NBUF, slot).start()
            _owrite(og, out_v, sem_o).start()
        return carry

    lax.fori_loop(0, NITER, _step, None)
    _owrite(2 * NITER - 2, out_a, sem_oa).wait()
    _owrite(2 * NITER - 1, out_b, sem_ob).wait()


def kernel(features, coarse_map):
    feat_flat = features.reshape(B * N, C)
    idx = coarse_map.reshape(NW, NCHUNK, ROWS)
    out = _pool(feat_flat, idx)
    return out.reshape(B, NPOINT, C)
